# baseline (speedup 1.0000x reference)
import jax
import jax.numpy as jnp
from jax.experimental import pallas as pl

_G = 25
_R = 625 // _G
_ER = 320000 // _G // 256


def _pool_body(x_ref, e_ref, o1, o2, o3, o4, g1, g2, g3, g4):
    v = x_ref[...]
    e = e_ref[...]
    g1[...] = e >> 1
    g2[...] = e >> 2
    g3[...] = e >> 3
    g4[...] = e >> 4

    def pool(t, groups):
        even = jnp.concatenate(
            [t[..., (2 * j) * 256:(2 * j + 1) * 256] for j in range(groups)],
            axis=-1)
        odd = jnp.concatenate(
            [t[..., (2 * j + 1) * 256:(2 * j + 2) * 256] for j in range(groups)],
            axis=-1)
        return (even + odd) * 0.5

    p1 = pool(v, 8)
    p2 = pool(p1, 4)
    p3 = pool(p2, 2)
    p4 = pool(p3, 1)
    o1[...] = p1
    o2[...] = p2
    o3[...] = p3
    o4[...] = p4


def kernel(x, edge_index):
    xr = x.reshape(_G, _R, 4096)
    er = edge_index.reshape(_G, _ER, 256)

    fspec = lambda shp: pl.BlockSpec((1,) + shp[1:], lambda i: (i, 0, 0))
    outs = pl.pallas_call(
        _pool_body,
        grid=(_G,),
        in_specs=[fspec((_G, _R, 4096)), fspec((_G, _ER, 256))],
        out_specs=[
            fspec((_G, _R, 2048)), fspec((_G, _R, 1024)),
            fspec((_G, _R, 512)), fspec((_G, _R, 256)),
            fspec((_G, _ER, 256)), fspec((_G, _ER, 256)),
            fspec((_G, _ER, 256)), fspec((_G, _ER, 256)),
        ],
        out_shape=[
            jax.ShapeDtypeStruct((_G, _R, 2048), jnp.float32),
            jax.ShapeDtypeStruct((_G, _R, 1024), jnp.float32),
            jax.ShapeDtypeStruct((_G, _R, 512), jnp.float32),
            jax.ShapeDtypeStruct((_G, _R, 256), jnp.float32),
            jax.ShapeDtypeStruct((_G, _ER, 256), jnp.int32),
            jax.ShapeDtypeStruct((_G, _ER, 256), jnp.int32),
            jax.ShapeDtypeStruct((_G, _ER, 256), jnp.int32),
            jax.ShapeDtypeStruct((_G, _ER, 256), jnp.int32),
        ],
    )(xr, er)
    p1, p2, p3, p4, f1, f2, f3, f4 = outs

    x1 = p1.reshape(5000, 256)
    x2 = p2.reshape(2500, 256)
    x3 = p3.reshape(1250, 256)
    x4 = p4.reshape(625, 256)
    e1 = f1.reshape(2, 160000)
    e2 = f2.reshape(2, 160000)
    e3 = f3.reshape(2, 160000)
    e4 = f4.reshape(2, 160000)
    return (x, x1, x2, x3, x4, edge_index, e1, e2, e3, e4)


# speedup vs baseline: 1.2421x; 1.2421x over previous
import jax
import jax.numpy as jnp
from jax.experimental import pallas as pl

_G = 1
_R = 625 // _G
_ER = 320000 // _G // 256


def _pool_body(x_ref, e_ref, o1, o2, o3, o4, g1, g2, g3, g4):
    v = x_ref[...]
    e = e_ref[...]
    g1[...] = e >> 1
    g2[...] = e >> 2
    g3[...] = e >> 3
    g4[...] = e >> 4

    def pool(t, groups):
        even = jnp.concatenate(
            [t[..., (2 * j) * 256:(2 * j + 1) * 256] for j in range(groups)],
            axis=-1)
        odd = jnp.concatenate(
            [t[..., (2 * j + 1) * 256:(2 * j + 2) * 256] for j in range(groups)],
            axis=-1)
        return (even + odd) * 0.5

    p1 = pool(v, 8)
    p2 = pool(p1, 4)
    p3 = pool(p2, 2)
    p4 = pool(p3, 1)
    o1[...] = p1
    o2[...] = p2
    o3[...] = p3
    o4[...] = p4


def kernel(x, edge_index):
    xr = x.reshape(_G, _R, 4096)
    er = edge_index.reshape(_G, _ER, 256)

    fspec = lambda shp: pl.BlockSpec((1,) + shp[1:], lambda i: (i, 0, 0))
    outs = pl.pallas_call(
        _pool_body,
        grid=(_G,),
        in_specs=[fspec((_G, _R, 4096)), fspec((_G, _ER, 256))],
        out_specs=[
            fspec((_G, _R, 2048)), fspec((_G, _R, 1024)),
            fspec((_G, _R, 512)), fspec((_G, _R, 256)),
            fspec((_G, _ER, 256)), fspec((_G, _ER, 256)),
            fspec((_G, _ER, 256)), fspec((_G, _ER, 256)),
        ],
        out_shape=[
            jax.ShapeDtypeStruct((_G, _R, 2048), jnp.float32),
            jax.ShapeDtypeStruct((_G, _R, 1024), jnp.float32),
            jax.ShapeDtypeStruct((_G, _R, 512), jnp.float32),
            jax.ShapeDtypeStruct((_G, _R, 256), jnp.float32),
            jax.ShapeDtypeStruct((_G, _ER, 256), jnp.int32),
            jax.ShapeDtypeStruct((_G, _ER, 256), jnp.int32),
            jax.ShapeDtypeStruct((_G, _ER, 256), jnp.int32),
            jax.ShapeDtypeStruct((_G, _ER, 256), jnp.int32),
        ],
    )(xr, er)
    p1, p2, p3, p4, f1, f2, f3, f4 = outs

    x1 = p1.reshape(5000, 256)
    x2 = p2.reshape(2500, 256)
    x3 = p3.reshape(1250, 256)
    x4 = p4.reshape(625, 256)
    e1 = f1.reshape(2, 160000)
    e2 = f2.reshape(2, 160000)
    e3 = f3.reshape(2, 160000)
    e4 = f4.reshape(2, 160000)
    return (x, x1, x2, x3, x4, edge_index, e1, e2, e3, e4)


# speedup vs baseline: 3.0343x; 2.4427x over previous
import jax
import jax.numpy as jnp
from jax.experimental import pallas as pl


def _body(x_ref, e_ref, o1, o2, o3, o4, g1, g2, g3, g4):
    e = e_ref[...]
    g1[...] = e >> 1
    g2[...] = e >> 2
    g3[...] = e >> 3
    g4[...] = e >> 4

    def pool(t):
        n = t.shape[0]
        m = t.reshape(n // 2, 512)
        return (m[:, :256] + m[:, 256:]) * 0.5

    p1 = pool(x_ref[...])
    p2 = pool(p1)
    p3 = pool(p2)
    p4 = pool(p3)
    o1[...] = p1
    o2[...] = p2
    o3[...] = p3
    o4[...] = p4


def kernel(x, edge_index):
    outs = pl.pallas_call(
        _body,
        out_shape=[
            jax.ShapeDtypeStruct((5000, 256), jnp.float32),
            jax.ShapeDtypeStruct((2500, 256), jnp.float32),
            jax.ShapeDtypeStruct((1250, 256), jnp.float32),
            jax.ShapeDtypeStruct((625, 256), jnp.float32),
            jax.ShapeDtypeStruct((2, 160000), jnp.int32),
            jax.ShapeDtypeStruct((2, 160000), jnp.int32),
            jax.ShapeDtypeStruct((2, 160000), jnp.int32),
            jax.ShapeDtypeStruct((2, 160000), jnp.int32),
        ],
    )(x, edge_index)
    x1, x2, x3, x4, e1, e2, e3, e4 = outs
    return (x, x1, x2, x3, x4, edge_index, e1, e2, e3, e4)


# speedup vs baseline: 3.9643x; 1.3065x over previous
import jax
import jax.numpy as jnp
from jax.experimental import pallas as pl


def _body(x_ref, e_ref, o0, o1, o2, o3, o4, g0, g1, g2, g3, g4):
    e = e_ref[...]
    g0[...] = e
    g1[...] = e >> 1
    g2[...] = e >> 2
    g3[...] = e >> 3
    g4[...] = e >> 4

    def pool(t):
        n = t.shape[0]
        m = t.reshape(n // 2, 512)
        return (m[:, :256] + m[:, 256:]) * 0.5

    v = x_ref[...]
    o0[...] = v
    p1 = pool(v)
    p2 = pool(p1)
    p3 = pool(p2)
    p4 = pool(p3)
    o1[...] = p1
    o2[...] = p2
    o3[...] = p3
    o4[...] = p4


def kernel(x, edge_index):
    outs = pl.pallas_call(
        _body,
        out_shape=[
            jax.ShapeDtypeStruct((10000, 256), jnp.float32),
            jax.ShapeDtypeStruct((5000, 256), jnp.float32),
            jax.ShapeDtypeStruct((2500, 256), jnp.float32),
            jax.ShapeDtypeStruct((1250, 256), jnp.float32),
            jax.ShapeDtypeStruct((625, 256), jnp.float32),
            jax.ShapeDtypeStruct((2, 160000), jnp.int32),
            jax.ShapeDtypeStruct((2, 160000), jnp.int32),
            jax.ShapeDtypeStruct((2, 160000), jnp.int32),
            jax.ShapeDtypeStruct((2, 160000), jnp.int32),
            jax.ShapeDtypeStruct((2, 160000), jnp.int32),
        ],
    )(x, edge_index)
    x0, x1, x2, x3, x4, e0, e1, e2, e3, e4 = outs
    return (x0, x1, x2, x3, x4, e0, e1, e2, e3, e4)
